# Initial kernel scaffold; baseline (speedup 1.0000x reference)
import jax
import jax.numpy as jnp
from jax.experimental import pallas as pl


def kernel(x, pe):
    raise NotImplementedError("write your pallas kernel here")


# speedup vs baseline: 1.6926x; 1.6926x over previous
import jax
import jax.numpy as jnp
from jax.experimental import pallas as pl

_S_BLK = 512


def _body(x_ref, pe_ref, o_ref):
    o_ref[...] = x_ref[...] + pe_ref[...]


def kernel(x, pe):
    B, S, E = x.shape
    grid = (S // _S_BLK, B)
    return pl.pallas_call(
        _body,
        grid=grid,
        in_specs=[
            pl.BlockSpec((1, _S_BLK, E), lambda i, b: (b, i, 0)),
            pl.BlockSpec((_S_BLK, E), lambda i, b: (i, 0)),
        ],
        out_specs=pl.BlockSpec((1, _S_BLK, E), lambda i, b: (b, i, 0)),
        out_shape=jax.ShapeDtypeStruct(x.shape, x.dtype),
    )(x, pe)


# speedup vs baseline: 1.8770x; 1.1089x over previous
import jax
import jax.numpy as jnp
from jax.experimental import pallas as pl

_S_BLK = 1024


def _body(x_ref, pe_ref, o_ref):
    o_ref[...] = x_ref[...] + pe_ref[...]


def kernel(x, pe):
    B, S, E = x.shape
    grid = (S // _S_BLK, B)
    return pl.pallas_call(
        _body,
        grid=grid,
        in_specs=[
            pl.BlockSpec((1, _S_BLK, E), lambda i, b: (b, i, 0)),
            pl.BlockSpec((_S_BLK, E), lambda i, b: (i, 0)),
        ],
        out_specs=pl.BlockSpec((1, _S_BLK, E), lambda i, b: (b, i, 0)),
        out_shape=jax.ShapeDtypeStruct(x.shape, x.dtype),
    )(x, pe)


# speedup vs baseline: 1.9932x; 1.0619x over previous
import jax
import jax.numpy as jnp
from jax.experimental import pallas as pl

_S_BLK = 2048


def _body(x_ref, pe_ref, o_ref):
    o_ref[...] = x_ref[...] + pe_ref[...]


def kernel(x, pe):
    B, S, E = x.shape
    grid = (S // _S_BLK, B)
    return pl.pallas_call(
        _body,
        grid=grid,
        in_specs=[
            pl.BlockSpec((1, _S_BLK, E), lambda i, b: (b, i, 0)),
            pl.BlockSpec((_S_BLK, E), lambda i, b: (i, 0)),
        ],
        out_specs=pl.BlockSpec((1, _S_BLK, E), lambda i, b: (b, i, 0)),
        out_shape=jax.ShapeDtypeStruct(x.shape, x.dtype),
    )(x, pe)
